# baseline (speedup 1.0000x reference)
import functools

import jax
import jax.numpy as jnp
from jax import lax
from jax.experimental import pallas as pl
from jax.experimental.pallas import tpu as pltpu
from jax.experimental.pallas import tpu_sc as plsc

_VOCAB = 1000000
_DIM = 64
_PAIR = 2 * _DIM
_BATCH = 16384
_LANES = 16

_NC = 2
_NS = 16
_NW = _NC * _NS
_BPW = _BATCH // _NW
_CHUNK = 128
_NCHUNK = _BPW // _CHUNK
_ROWS = _BATCH // 128
_NBUF = 2

_PC = 80
_NPC = (_VOCAB // 2) // _PC
_PK = (_NPC + _NW - 1) // _NW


def _lane_permute(x, idx):
    return lax.gather(
        x, idx[:, None],
        lax.GatherDimensionNumbers(
            offset_dims=(), collapsed_slice_dims=(0,), start_index_map=(0,)),
        slice_sizes=(1,),
        mode=lax.GatherScatterMode.PROMISE_IN_BOUNDS)


def _sc_pack_body(tt_hbm, ct_hbm, ts_hbm, cs_hbm,
                  tin, cin, tout, cout, sem_in, sem_out):
    wid = lax.axis_index("s") * _NC + lax.axis_index("c")

    def in_copies(cid, slot):
        src = cid * (2 * _PC)
        return ((tt_hbm.at[pl.ds(src, 2 * _PC)], tin.at[slot]),
                (ct_hbm.at[pl.ds(src, 2 * _PC)], cin.at[slot]))

    def out_copies(cid, slot):
        dst = cid * _PC
        return ((tout.at[slot], ts_hbm.at[pl.ds(dst, _PC)]),
                (cout.at[slot], cs_hbm.at[pl.ds(dst, _PC)]))

    def repack(k):
        slot = lax.rem(k, _NBUF)
        for r2 in range(_PC):
            for v in range(_PAIR // _LANES):
                src_r = 2 * r2 + (v >> 2)
                src_c = (v & 3) * _LANES
                tout[slot, r2, pl.ds(v * _LANES, _LANES)] = (
                    tin[slot, src_r, pl.ds(src_c, _LANES)])
                cout[slot, r2, pl.ds(v * _LANES, _LANES)] = (
                    cin[slot, src_r, pl.ds(src_c, _LANES)])

    def fire_in(k):
        cid = wid + k * _NW

        @pl.when(cid < _NPC)
        def _():
            for cp in in_copies(cid, lax.rem(k, _NBUF)):
                pltpu.async_copy(*cp, sem_in)

    def wait_in(k):
        cid = wid + k * _NW

        @pl.when(cid < _NPC)
        def _():
            for cp in in_copies(cid, lax.rem(k, _NBUF)):
                pltpu.make_async_copy(*cp, sem_in).wait()

    def fire_out(k):
        cid = wid + k * _NW

        @pl.when(cid < _NPC)
        def _():
            for cp in out_copies(cid, lax.rem(k, _NBUF)):
                pltpu.async_copy(*cp, sem_out)

    def wait_out(k):
        cid = wid + k * _NW

        @pl.when((cid < _NPC) & (k >= 0))
        def _():
            for cp in out_copies(cid, lax.rem(k, _NBUF)):
                pltpu.make_async_copy(*cp, sem_out).wait()

    fire_in(jnp.int32(0))

    def body(k, _):
        wait_in(k)
        wait_out(k - 1)
        fire_in(k + 1)

        @pl.when(wid + k * _NW < _NPC)
        def _():
            repack(k)

        fire_out(k)
        return 0

    lax.fori_loop(0, _PK, body, 0)
    wait_out(jnp.int32(_PK - 1))


def _sc_scores_body(t_hbm, c_hbm, tt_hbm, ct_hbm, out_hbm,
                    tidx, cidx, tidx_g, cidx_g, trows, crows, scores, sem):
    wid = lax.axis_index("s") * _NC + lax.axis_index("c")
    base = wid * _NCHUNK

    pltpu.sync_copy(t_hbm.at[pl.ds(base, _NCHUNK)], tidx)
    pltpu.sync_copy(c_hbm.at[pl.ds(base, _NCHUNK)], cidx)

    for ci in range(_NCHUNK):
        for v in range(_CHUNK // _LANES):
            sl = pl.ds(v * _LANES, _LANES)
            tidx_g[ci, sl] = tidx[ci, sl] >> 1
            cidx_g[ci, sl] = cidx[ci, sl] >> 1

    def fire(ci, slot):
        return (pltpu.async_copy(tt_hbm.at[tidx_g.at[ci]], trows.at[slot],
                                 sem),
                pltpu.async_copy(ct_hbm.at[cidx_g.at[ci]], crows.at[slot],
                                 sem))

    lane = lax.iota(jnp.int32, _LANES)
    stages = [(lane ^ h, (lane & h) == 0) for h in (8, 4, 2, 1)]
    bitrev = (((lane & 1) << 3) | ((lane & 2) << 1)
              | ((lane & 4) >> 1) | ((lane & 8) >> 3))

    def merge(a, b, perm_h, mask_h):
        u = a + _lane_permute(a, perm_h)
        v = b + _lane_permute(b, perm_h)
        return jnp.where(mask_h, u, v)

    def tree(vecs):
        for perm_h, mask_h in stages:
            vecs = [merge(vecs[i], vecs[i + 1], perm_h, mask_h)
                    for i in range(0, len(vecs), 2)]
        return _lane_permute(vecs[0], bitrev)

    pending = fire(0, 0)
    for ci in range(_NCHUNK):
        slot = ci % _NBUF
        pending[0].wait()
        pending[1].wait()
        if ci + 1 < _NCHUNK:
            pending = fire(ci + 1, (ci + 1) % _NBUF)

        for g in range(_CHUNK // _LANES):
            sl = pl.ds(g * _LANES, _LANES)
            tvec = tidx[ci, sl]
            cvec = cidx[ci, sl]
            vecs = []
            for r in range(_LANES):
                j = g * _LANES + r
                toff = (tvec[r] & 1) * _DIM
                coff = (cvec[r] & 1) * _DIM
                acc = (trows[slot, j, pl.ds(toff, _LANES)]
                       * crows[slot, j, pl.ds(coff, _LANES)])
                for k in range(1, _DIM // _LANES):
                    acc = acc + (
                        trows[slot, j, pl.ds(toff + k * _LANES, _LANES)]
                        * crows[slot, j, pl.ds(coff + k * _LANES, _LANES)])
                vecs.append(acc)
            scores[ci, sl] = tree(vecs)

    pltpu.sync_copy(scores, out_hbm.at[pl.ds(base, _NCHUNK)])


@jax.jit
def _sc_forward(t_idx, c_idx, target_table, context_table):
    mesh = plsc.VectorSubcoreMesh(core_axis_name="c", subcore_axis_name="s")

    pack = functools.partial(
        pl.kernel,
        mesh=mesh,
        out_type=(jax.ShapeDtypeStruct((_VOCAB // 2, _PAIR), jnp.float32),
                  jax.ShapeDtypeStruct((_VOCAB // 2, _PAIR), jnp.float32)),
        scratch_types=[
            pltpu.VMEM((_NBUF, 2 * _PC, _DIM), jnp.float32),
            pltpu.VMEM((_NBUF, 2 * _PC, _DIM), jnp.float32),
            pltpu.VMEM((_NBUF, _PC, _PAIR), jnp.float32),
            pltpu.VMEM((_NBUF, _PC, _PAIR), jnp.float32),
            pltpu.SemaphoreType.DMA,
            pltpu.SemaphoreType.DMA,
        ],
    )(_sc_pack_body)
    tt_pairs, ct_pairs = pack(target_table, context_table)

    score = functools.partial(
        pl.kernel,
        mesh=mesh,
        out_type=jax.ShapeDtypeStruct((_ROWS, 128), jnp.float32),
        scratch_types=[
            pltpu.VMEM((_NCHUNK, _CHUNK), jnp.int32),
            pltpu.VMEM((_NCHUNK, _CHUNK), jnp.int32),
            pltpu.VMEM((_NCHUNK, _CHUNK), jnp.int32),
            pltpu.VMEM((_NCHUNK, _CHUNK), jnp.int32),
            pltpu.VMEM((_NBUF, _CHUNK, _PAIR), jnp.float32),
            pltpu.VMEM((_NBUF, _CHUNK, _PAIR), jnp.float32),
            pltpu.VMEM((_NCHUNK, _CHUNK), jnp.float32),
            pltpu.SemaphoreType.DMA,
        ],
    )(_sc_scores_body)
    return score(t_idx, c_idx, tt_pairs, ct_pairs)


def kernel(target, context, target_table, context_table):
    t_idx = target.astype(jnp.int32).reshape(_ROWS, 128)
    c_idx = context.astype(jnp.int32).reshape(_ROWS, 128)
    out = _sc_forward(t_idx, c_idx, target_table, context_table)
    return out.reshape(_BATCH)


# speedup vs baseline: 2.6880x; 2.6880x over previous
import functools

import jax
import jax.numpy as jnp
from jax import lax
from jax.experimental import pallas as pl
from jax.experimental.pallas import tpu as pltpu
from jax.experimental.pallas import tpu_sc as plsc

_VOCAB = 1000000
_DIM = 64
_BATCH = 16384
_LANES = 16

_NC = 2
_NS = 16
_NW = _NC * _NS
_BPW = _BATCH // _NW
_CHUNK = 32
_NCHUNK = _BPW // _CHUNK
_IROWS = _BPW // 128
_ROWS = _BATCH // 128
_NBUF = 8
_NSEM = 4


def _lane_permute(x, idx):
    return lax.gather(
        x, idx[:, None],
        lax.GatherDimensionNumbers(
            offset_dims=(), collapsed_slice_dims=(0,), start_index_map=(0,)),
        slice_sizes=(1,),
        mode=lax.GatherScatterMode.PROMISE_IN_BOUNDS)


def _sc_body(t_hbm, c_hbm, tt_hbm, ct_hbm, out_hbm,
             tidx_v, cidx_v, trows, crows, scores, *sems):
    wid = lax.axis_index("s") * _NC + lax.axis_index("c")
    base = wid * _IROWS

    pltpu.sync_copy(t_hbm.at[pl.ds(base, _IROWS)], tidx_v)
    pltpu.sync_copy(c_hbm.at[pl.ds(base, _IROWS)], cidx_v)

    def chunk_scalars(ci):
        r, c0 = ci >> 2, (ci & (128 // _CHUNK - 1)) * _CHUNK
        tis, cis = [], []
        for v in range(_CHUNK // _LANES):
            tvec = tidx_v[r, pl.ds(c0 + v * _LANES, _LANES)]
            cvec = cidx_v[r, pl.ds(c0 + v * _LANES, _LANES)]
            tis += [tvec[l] for l in range(_LANES)]
            cis += [cvec[l] for l in range(_LANES)]
        return tis, cis

    def row_copies(ti, ci_, slot, j):
        t_cp = (tt_hbm.at[pl.ds(ti, 1)], trows.at[slot, pl.ds(j, 1)])
        c_cp = (ct_hbm.at[pl.ds(ci_, 1)], crows.at[slot, pl.ds(j, 1)])
        return t_cp, c_cp

    def fire(ci, slot):
        tis, cis = chunk_scalars(ci)
        for j in range(_CHUNK):
            t_cp, c_cp = row_copies(tis[j], cis[j], slot, j)
            pltpu.async_copy(*t_cp, sems[j % _NSEM])
            pltpu.async_copy(*c_cp, sems[j % _NSEM])

    def drain(ci, slot):
        tis, cis = chunk_scalars(ci)
        for j in range(_CHUNK):
            t_cp, c_cp = row_copies(tis[j], cis[j], slot, j)
            pltpu.make_async_copy(*t_cp, sems[j % _NSEM]).wait()
            pltpu.make_async_copy(*c_cp, sems[j % _NSEM]).wait()

    lane = lax.iota(jnp.int32, _LANES)
    stages = [(lane ^ h, (lane & h) == 0) for h in (8, 4, 2, 1)]
    bitrev = (((lane & 1) << 3) | ((lane & 2) << 1)
              | ((lane & 4) >> 1) | ((lane & 8) >> 3))

    def merge(a, b, perm_h, mask_h):
        u = a + _lane_permute(a, perm_h)
        v = b + _lane_permute(b, perm_h)
        return jnp.where(mask_h, u, v)

    def tree(vecs):
        for perm_h, mask_h in stages:
            vecs = [merge(vecs[i], vecs[i + 1], perm_h, mask_h)
                    for i in range(0, len(vecs), 2)]
        return _lane_permute(vecs[0], bitrev)

    for ci in range(_NBUF - 1):
        fire(ci, ci)

    def chunk_body(ci, _):
        slot = lax.rem(ci, _NBUF)
        drain(ci, slot)

        @pl.when(ci + _NBUF - 1 < _NCHUNK)
        def _():
            nxt = ci + _NBUF - 1
            fire(nxt, lax.rem(nxt, _NBUF))

        for g in range(_CHUNK // _LANES):
            vecs = []
            for r in range(_LANES):
                j = g * _LANES + r
                acc = (trows[slot, j, pl.ds(0, _LANES)]
                       * crows[slot, j, pl.ds(0, _LANES)])
                for k in range(1, _DIM // _LANES):
                    acc = acc + (trows[slot, j, pl.ds(k * _LANES, _LANES)]
                                 * crows[slot, j, pl.ds(k * _LANES, _LANES)])
                vecs.append(acc)
            totals = tree(vecs)
            flat = ci * _CHUNK + g * _LANES
            scores[flat >> 7, pl.ds(flat & 127, _LANES)] = totals
        return 0

    lax.fori_loop(0, _NCHUNK, chunk_body, 0)

    pltpu.sync_copy(scores, out_hbm.at[pl.ds(base, _IROWS)])


@jax.jit
def _sc_scores(t_idx, c_idx, target_table, context_table):
    mesh = plsc.VectorSubcoreMesh(core_axis_name="c", subcore_axis_name="s")
    k = functools.partial(
        pl.kernel,
        mesh=mesh,
        out_type=jax.ShapeDtypeStruct((_ROWS, 128), jnp.float32),
        scratch_types=[
            pltpu.VMEM((_IROWS, 128), jnp.int32),
            pltpu.VMEM((_IROWS, 128), jnp.int32),
            pltpu.VMEM((_NBUF, _CHUNK, _DIM), jnp.float32),
            pltpu.VMEM((_NBUF, _CHUNK, _DIM), jnp.float32),
            pltpu.VMEM((_IROWS, 128), jnp.float32),
        ] + [pltpu.SemaphoreType.DMA] * _NSEM,
    )(_sc_body)
    return k(t_idx, c_idx, target_table, context_table)


def kernel(target, context, target_table, context_table):
    t_idx = target.astype(jnp.int32).reshape(_ROWS, 128)
    c_idx = context.astype(jnp.int32).reshape(_ROWS, 128)
    out = _sc_scores(t_idx, c_idx, target_table, context_table)
    return out.reshape(_BATCH)
